# initial kernel scaffold (unmeasured)
import jax
import jax.numpy as jnp
from jax import lax
from jax.experimental import pallas as pl
from jax.experimental.pallas import tpu as pltpu

N_DEV = 16
SQ = 2048
SKV = 2048
H_PER = 8
DH = 128
D_MODEL = 1024
D_HEADS = H_PER * DH
SCALE = 0.08838834764831843
CHUNK = SQ // N_DEV
QB = 512
N_HOP = N_DEV - 1


def kernel(x, Wq, K_ext, V_ext, Wo):
    pos = lax.axis_index("i")
    wq_s = lax.dynamic_slice(Wq, (0, pos * D_HEADS), (D_MODEL, D_HEADS))
    wo_s = lax.dynamic_slice(Wo, (pos * D_HEADS, 0), (D_HEADS, D_MODEL))
    x2 = x[0]
    k = K_ext[0]
    v = V_ext[0]

    def body(x_ref, wq_ref, k_ref, v_ref, wo_ref, out_ref,
             comm_ref, send_sems, recv_sems, phase_sem):
        p = lax.axis_index("i")
        left = lax.rem(p + N_DEV - 1, N_DEV)
        right = lax.rem(p + 1, N_DEV)

        def mod(a):
            return lax.rem(a + 2 * N_DEV, N_DEV)

        barrier_sem = pltpu.get_barrier_semaphore()
        for nbr in (left, right):
            pl.semaphore_signal(barrier_sem, inc=1, device_id=(nbr,),
                                device_id_type=pl.DeviceIdType.MESH)
        pl.semaphore_wait(barrier_sem, 2)

        for qb in range(SQ // QB):
            x_blk = x_ref[qb * QB:(qb + 1) * QB, :]
            qi = lax.broadcasted_iota(jnp.int32, (QB, SKV), 0) + qb * QB
            ki = lax.broadcasted_iota(jnp.int32, (QB, SKV), 1)
            mask = (jnp.abs(qi - ki) <= 128) | (ki < 32) | (qi < 32)
            acc = jnp.zeros((QB, D_MODEL), jnp.float32)
            for h in range(H_PER):
                qh = jnp.dot(x_blk, wq_ref[:, h * DH:(h + 1) * DH],
                             preferred_element_type=jnp.float32)
                kh = k_ref[:, h, :]
                vh = v_ref[:, h, :]
                scores = lax.dot_general(
                    qh, kh, (((1,), (1,)), ((), ())),
                    preferred_element_type=jnp.float32) * SCALE
                scores = jnp.where(mask, scores, jnp.float32(-1e9))
                m = jnp.max(scores, axis=1, keepdims=True)
                w = jnp.exp(scores - m)
                w = w / jnp.sum(w, axis=1, keepdims=True)
                ctx = jnp.dot(w, vh, preferred_element_type=jnp.float32)
                acc = acc + jnp.dot(ctx, wo_ref[h * DH:(h + 1) * DH, :],
                                    preferred_element_type=jnp.float32)
            out_ref[qb * QB:(qb + 1) * QB, :] = acc

        for s in range(N_HOP):
            sc = mod(p - s)
            rc = mod(p - s - 1)
            rdma = pltpu.make_async_remote_copy(
                src_ref=out_ref.at[pl.ds(sc * CHUNK, CHUNK), :],
                dst_ref=comm_ref.at[s],
                send_sem=send_sems.at[s],
                recv_sem=recv_sems.at[s],
                device_id=(right,),
                device_id_type=pl.DeviceIdType.MESH,
            )
            rdma.start()
            rdma.wait()
            out_ref[pl.ds(rc * CHUNK, CHUNK), :] += comm_ref[s]

        pl.semaphore_signal(phase_sem, inc=1, device_id=(left,),
                            device_id_type=pl.DeviceIdType.MESH)
        pl.semaphore_wait(phase_sem, 1)

        for s in range(N_HOP):
            gc = mod(p + 1 - s)
            rdma = pltpu.make_async_remote_copy(
                src_ref=out_ref.at[pl.ds(gc * CHUNK, CHUNK), :],
                dst_ref=out_ref.at[pl.ds(gc * CHUNK, CHUNK), :],
                send_sem=send_sems.at[N_HOP + s],
                recv_sem=recv_sems.at[N_HOP + s],
                device_id=(right,),
                device_id_type=pl.DeviceIdType.MESH,
            )
            rdma.start()
            rdma.wait()

    out = pl.pallas_call(
        body,
        out_shape=jax.ShapeDtypeStruct((SQ, D_MODEL), jnp.float32),
        in_specs=[pl.BlockSpec(memory_space=pltpu.VMEM)] * 5,
        out_specs=pl.BlockSpec(memory_space=pltpu.VMEM),
        scratch_shapes=[
            pltpu.VMEM((N_HOP, CHUNK, D_MODEL), jnp.float32),
            pltpu.SemaphoreType.DMA((2 * N_HOP,)),
            pltpu.SemaphoreType.DMA((2 * N_HOP,)),
            pltpu.SemaphoreType.REGULAR,
        ],
        compiler_params=pltpu.CompilerParams(collective_id=0),
    )(x2, wq_s, k, v, wo_s)
    return out[None]


# baseline (device time: 390589 ns/iter reference)
import jax
import jax.numpy as jnp
from jax import lax
from jax.experimental import pallas as pl
from jax.experimental.pallas import tpu as pltpu

N_DEV = 16
SQ = 2048
SKV = 2048
H_PER = 8
DH = 128
D_MODEL = 1024
D_HEADS = H_PER * DH
SCALE = 0.08838834764831843
CHUNK = SQ // N_DEV
QB = 512
N_HOP = N_DEV - 1


def kernel(x, Wq, K_ext, V_ext, Wo):
    pos = lax.axis_index("i")
    wq_s = lax.dynamic_slice(Wq, (0, pos * D_HEADS), (D_MODEL, D_HEADS))
    wo_s = lax.dynamic_slice(Wo, (pos * D_HEADS, 0), (D_HEADS, D_MODEL))
    x2 = x[0]
    k = jnp.transpose(K_ext[0], (1, 0, 2))
    v = jnp.transpose(V_ext[0], (1, 0, 2))

    def body(x_ref, wq_ref, k_ref, v_ref, wo_ref, out_ref,
             comm_ref, send_sems, recv_sems, phase_sem):
        p = lax.axis_index("i")
        left = lax.rem(p + N_DEV - 1, N_DEV)
        right = lax.rem(p + 1, N_DEV)

        def mod(a):
            return lax.rem(a + 2 * N_DEV, N_DEV)

        barrier_sem = pltpu.get_barrier_semaphore()
        for nbr in (left, right):
            pl.semaphore_signal(barrier_sem, inc=1, device_id=(nbr,),
                                device_id_type=pl.DeviceIdType.MESH)
        pl.semaphore_wait(barrier_sem, 2)

        for qb in range(SQ // QB):
            x_blk = x_ref[qb * QB:(qb + 1) * QB, :]
            qi = lax.broadcasted_iota(jnp.int32, (QB, SKV), 0) + qb * QB
            ki = lax.broadcasted_iota(jnp.int32, (QB, SKV), 1)
            mask = (jnp.abs(qi - ki) <= 128) | (ki < 32) | (qi < 32)

            def head_step(h, acc):
                qh = jnp.dot(x_blk, wq_ref[:, pl.ds(h * DH, DH)],
                             preferred_element_type=jnp.float32)
                kh = k_ref[h]
                vh = v_ref[h]
                scores = lax.dot_general(
                    qh, kh, (((1,), (1,)), ((), ())),
                    preferred_element_type=jnp.float32) * SCALE
                scores = jnp.where(mask, scores, jnp.float32(-1e9))
                m = jnp.max(scores, axis=1, keepdims=True)
                w = jnp.exp(scores - m)
                w = w / jnp.sum(w, axis=1, keepdims=True)
                ctx = jnp.dot(w, vh, preferred_element_type=jnp.float32)
                woh = wo_ref[pl.ds(h * DH, DH), :]
                return acc + jnp.dot(ctx, woh,
                                     preferred_element_type=jnp.float32)

            acc = lax.fori_loop(0, H_PER, head_step,
                                jnp.zeros((QB, D_MODEL), jnp.float32))
            out_ref[qb * QB:(qb + 1) * QB, :] = acc

        for s in range(N_HOP):
            sc = mod(p - s)
            rc = mod(p - s - 1)
            rdma = pltpu.make_async_remote_copy(
                src_ref=out_ref.at[pl.ds(sc * CHUNK, CHUNK), :],
                dst_ref=comm_ref.at[s],
                send_sem=send_sems.at[s],
                recv_sem=recv_sems.at[s],
                device_id=(right,),
                device_id_type=pl.DeviceIdType.MESH,
            )
            rdma.start()
            rdma.wait()
            out_ref[pl.ds(rc * CHUNK, CHUNK), :] += comm_ref[s]

        pl.semaphore_signal(phase_sem, inc=1, device_id=(left,),
                            device_id_type=pl.DeviceIdType.MESH)
        pl.semaphore_wait(phase_sem, 1)

        for s in range(N_HOP):
            gc = mod(p + 1 - s)
            rdma = pltpu.make_async_remote_copy(
                src_ref=out_ref.at[pl.ds(gc * CHUNK, CHUNK), :],
                dst_ref=out_ref.at[pl.ds(gc * CHUNK, CHUNK), :],
                send_sem=send_sems.at[N_HOP + s],
                recv_sem=recv_sems.at[N_HOP + s],
                device_id=(right,),
                device_id_type=pl.DeviceIdType.MESH,
            )
            rdma.start()
            rdma.wait()

    out = pl.pallas_call(
        body,
        out_shape=jax.ShapeDtypeStruct((SQ, D_MODEL), jnp.float32),
        in_specs=[pl.BlockSpec(memory_space=pltpu.VMEM)] * 5,
        out_specs=pl.BlockSpec(memory_space=pltpu.VMEM),
        scratch_shapes=[
            pltpu.VMEM((N_HOP, CHUNK, D_MODEL), jnp.float32),
            pltpu.SemaphoreType.DMA((2 * N_HOP,)),
            pltpu.SemaphoreType.DMA((2 * N_HOP,)),
            pltpu.SemaphoreType.REGULAR,
        ],
        compiler_params=pltpu.CompilerParams(
            collective_id=0,
            vmem_limit_bytes=100 * 1024 * 1024,
        ),
    )(x2, wq_s, k, v, wo_s)
    return out[None]


# device time: 294826 ns/iter; 1.3248x vs baseline; 1.3248x over previous
import jax
import jax.numpy as jnp
from jax import lax
from jax.experimental import pallas as pl
from jax.experimental.pallas import tpu as pltpu

N_DEV = 16
SQ = 2048
SKV = 2048
H_PER = 8
DH = 128
D_MODEL = 1024
D_HEADS = H_PER * DH
SCALE = 0.08838834764831843
CHUNK = SQ // N_DEV
N_HOP = N_DEV - 1


def kernel(x, Wq, K_ext, V_ext, Wo):
    pos = lax.axis_index("i")
    wq_s = lax.dynamic_slice(Wq, (0, pos * D_HEADS), (D_MODEL, D_HEADS))
    wo_s = lax.dynamic_slice(Wo, (pos * D_HEADS, 0), (D_HEADS, D_MODEL))
    x2 = x[0]
    k = jnp.transpose(K_ext[0], (1, 0, 2))
    v = jnp.transpose(V_ext[0], (1, 0, 2))

    def body(x_ref, wq_ref, k_ref, v_ref, wo_ref, out_ref,
             comm_ref, ag_ref, send_sems, recv_sems):
        p = lax.axis_index("i")
        left = lax.rem(p + N_DEV - 1, N_DEV)
        right = lax.rem(p + 1, N_DEV)

        def mod(a):
            return lax.rem(a + 2 * N_DEV, N_DEV)

        def row(c):
            return pl.ds(c * CHUNK, CHUNK)

        barrier_sem = pltpu.get_barrier_semaphore()
        for nbr in (left, right):
            pl.semaphore_signal(barrier_sem, inc=1, device_id=(nbr,),
                                device_id_type=pl.DeviceIdType.MESH)
        pl.semaphore_wait(barrier_sem, 2)

        def chunk_partial(c):
            r0 = c * CHUNK
            x_blk = x_ref[row(c), :]
            qi = lax.broadcasted_iota(jnp.int32, (CHUNK, SKV), 0) + r0
            ki = lax.broadcasted_iota(jnp.int32, (CHUNK, SKV), 1)
            mask = (jnp.abs(qi - ki) <= 128) | (ki < 32) | (qi < 32)

            def head_step(h, acc):
                qh = jnp.dot(x_blk, wq_ref[:, pl.ds(h * DH, DH)],
                             preferred_element_type=jnp.float32)
                scores = lax.dot_general(
                    qh, k_ref[h], (((1,), (1,)), ((), ())),
                    preferred_element_type=jnp.float32) * SCALE
                scores = jnp.where(mask, scores, jnp.float32(-1e9))
                m = jnp.max(scores, axis=1, keepdims=True)
                w = jnp.exp(scores - m)
                w = w / jnp.sum(w, axis=1, keepdims=True)
                ctx = jnp.dot(w, v_ref[h], preferred_element_type=jnp.float32)
                return acc + jnp.dot(ctx, wo_ref[pl.ds(h * DH, DH), :],
                                     preferred_element_type=jnp.float32)

            return lax.fori_loop(0, H_PER, head_step,
                                 jnp.zeros((CHUNK, D_MODEL), jnp.float32))

        def rs_send(s, c):
            return pltpu.make_async_remote_copy(
                src_ref=out_ref.at[row(c), :],
                dst_ref=comm_ref.at[s],
                send_sem=send_sems.at[s],
                recv_sem=recv_sems.at[s],
                device_id=(right,),
                device_id_type=pl.DeviceIdType.MESH,
            )

        def rs_step(s, carry):
            c = mod(p - s)
            val = chunk_partial(c)
            out_ref[row(c), :] = val

            @pl.when(s > 0)
            def _():
                rs_send(s - 1, c).wait_recv()
                out_ref[row(c), :] += comm_ref[s - 1]

            @pl.when(s < N_DEV - 1)
            def _():
                rs_send(s, c).start()

            return carry

        lax.fori_loop(0, N_DEV, rs_step, 0)
        for s in range(N_HOP):
            rs_send(s, mod(p - s)).wait_send()

        own = mod(p + 1)
        ag_ref[own] = out_ref[row(own), :].astype(jnp.bfloat16)

        def ag_copy(s):
            g = mod(p + 1 - s)
            return pltpu.make_async_remote_copy(
                src_ref=ag_ref.at[g],
                dst_ref=ag_ref.at[g],
                send_sem=send_sems.at[N_HOP + s],
                recv_sem=recv_sems.at[N_HOP + s],
                device_id=(right,),
                device_id_type=pl.DeviceIdType.MESH,
            )

        def ag_recv(s):
            g = mod(p - s)
            return pltpu.make_async_remote_copy(
                src_ref=ag_ref.at[g],
                dst_ref=ag_ref.at[g],
                send_sem=send_sems.at[N_HOP + s],
                recv_sem=recv_sems.at[N_HOP + s],
                device_id=(left,),
                device_id_type=pl.DeviceIdType.MESH,
            )

        ag_copy(0).start()
        for s in range(1, N_HOP):
            ag_recv(s - 1).wait_recv()
            ag_copy(s).start()
            g = mod(p - s + 1)
            out_ref[row(g), :] = ag_ref[g].astype(jnp.float32)
        ag_recv(N_HOP - 1).wait_recv()
        g = mod(p - N_HOP + 1)
        out_ref[row(g), :] = ag_ref[g].astype(jnp.float32)
        for s in range(N_HOP):
            ag_copy(s).wait_send()

    out = pl.pallas_call(
        body,
        out_shape=jax.ShapeDtypeStruct((SQ, D_MODEL), jnp.float32),
        in_specs=[pl.BlockSpec(memory_space=pltpu.VMEM)] * 5,
        out_specs=pl.BlockSpec(memory_space=pltpu.VMEM),
        scratch_shapes=[
            pltpu.VMEM((N_HOP, CHUNK, D_MODEL), jnp.float32),
            pltpu.VMEM((N_DEV, CHUNK, D_MODEL), jnp.bfloat16),
            pltpu.SemaphoreType.DMA((2 * N_HOP,)),
            pltpu.SemaphoreType.DMA((2 * N_HOP,)),
        ],
        compiler_params=pltpu.CompilerParams(
            collective_id=0,
            vmem_limit_bytes=100 * 1024 * 1024,
        ),
    )(x2, wq_s, k, v, wo_s)
    return out[None]


# device time: 248473 ns/iter; 1.5720x vs baseline; 1.1866x over previous
import jax
import jax.numpy as jnp
from jax import lax
from jax.experimental import pallas as pl
from jax.experimental.pallas import tpu as pltpu

N_DEV = 16
SQ = 2048
SKV = 2048
H_PER = 8
DH = 128
D_MODEL = 1024
D_HEADS = H_PER * DH
SCALE = 0.08838834764831843
CHUNK = SQ // N_DEV
N_HOP = N_DEV - 1
WIN = 384
GLB = 128


def kernel(x, Wq, K_ext, V_ext, Wo):
    pos = lax.axis_index("i")
    wq_s = lax.dynamic_slice(Wq, (0, pos * D_HEADS),
                             (D_MODEL, D_HEADS)).astype(jnp.bfloat16)
    wo_s = lax.dynamic_slice(Wo, (pos * D_HEADS, 0),
                             (D_HEADS, D_MODEL)).astype(jnp.bfloat16)
    x2 = x[0].astype(jnp.bfloat16)
    k = jnp.transpose(K_ext[0], (1, 0, 2)).astype(jnp.bfloat16)
    v = jnp.transpose(V_ext[0], (1, 0, 2)).astype(jnp.bfloat16)

    def body(x_ref, wq_ref, k_ref, v_ref, wo_ref, out_ref,
             comm_ref, ag_ref, send_sems, recv_sems):
        p = lax.axis_index("i")
        left = lax.rem(p + N_DEV - 1, N_DEV)
        right = lax.rem(p + 1, N_DEV)

        def mod(a):
            return lax.rem(a + 2 * N_DEV, N_DEV)

        def row(c):
            return pl.ds(c * CHUNK, CHUNK)

        barrier_sem = pltpu.get_barrier_semaphore()
        for nbr in (left, right):
            pl.semaphore_signal(barrier_sem, inc=1, device_id=(nbr,),
                                device_id_type=pl.DeviceIdType.MESH)
        pl.semaphore_wait(barrier_sem, 2)

        def chunk_partial(c):
            r0 = c * CHUNK
            x_blk = x_ref[row(c), :]
            qi = lax.broadcasted_iota(jnp.int32, (CHUNK, SKV), 0) + r0

            def qh_of(h):
                return jnp.dot(x_blk, wq_ref[:, pl.ds(h * DH, DH)],
                               preferred_element_type=jnp.float32
                               ).astype(jnp.bfloat16)

            def dense_branch(_):
                ki = lax.broadcasted_iota(jnp.int32, (CHUNK, SKV), 1)
                mask = (jnp.abs(qi - ki) <= 128) | (ki < 32) | (qi < 32)

                def head_step(h, acc):
                    scores = lax.dot_general(
                        qh_of(h), k_ref[h], (((1,), (1,)), ((), ())),
                        preferred_element_type=jnp.float32) * SCALE
                    scores = jnp.where(mask, scores, jnp.float32(-1e9))
                    m = jnp.max(scores, axis=1, keepdims=True)
                    w = jnp.exp(scores - m)
                    w = w / jnp.sum(w, axis=1, keepdims=True)
                    ctx = jnp.dot(w.astype(jnp.bfloat16), v_ref[h],
                                  preferred_element_type=jnp.float32)
                    return acc + jnp.dot(ctx.astype(jnp.bfloat16),
                                         wo_ref[pl.ds(h * DH, DH), :],
                                         preferred_element_type=jnp.float32)

                return lax.fori_loop(0, H_PER, head_step,
                                     jnp.zeros((CHUNK, D_MODEL), jnp.float32))

            def window_branch(_):
                ws = lax.min((c - 1) * CHUNK, SKV - WIN)
                qi_w = qi[:, :WIN]
                ki_w = lax.broadcasted_iota(jnp.int32, (CHUNK, WIN), 1) + ws
                mask_w = (jnp.abs(qi_w - ki_w) <= 128) | (ki_w < 32)
                kg = lax.broadcasted_iota(jnp.int32, (CHUNK, GLB), 1)
                mask_g = (kg < 32) & (kg < ws)

                def head_step(h, acc):
                    qh = qh_of(h)
                    k_win = k_ref[h, pl.ds(ws, WIN), :]
                    sw = lax.dot_general(
                        qh, k_win, (((1,), (1,)), ((), ())),
                        preferred_element_type=jnp.float32) * SCALE
                    sg = lax.dot_general(
                        qh, k_ref[h, :GLB, :], (((1,), (1,)), ((), ())),
                        preferred_element_type=jnp.float32) * SCALE
                    sw = jnp.where(mask_w, sw, jnp.float32(-1e9))
                    sg = jnp.where(mask_g, sg, jnp.float32(-1e9))
                    m = jnp.maximum(jnp.max(sw, axis=1, keepdims=True),
                                    jnp.max(sg, axis=1, keepdims=True))
                    ww = jnp.exp(sw - m)
                    wg = jnp.exp(sg - m)
                    denom = (jnp.sum(ww, axis=1, keepdims=True)
                             + jnp.sum(wg, axis=1, keepdims=True))
                    ctx = (jnp.dot(ww.astype(jnp.bfloat16),
                                   v_ref[h, pl.ds(ws, WIN), :],
                                   preferred_element_type=jnp.float32)
                           + jnp.dot(wg.astype(jnp.bfloat16),
                                     v_ref[h, :GLB, :],
                                     preferred_element_type=jnp.float32))
                    ctx = ctx / denom
                    return acc + jnp.dot(ctx.astype(jnp.bfloat16),
                                         wo_ref[pl.ds(h * DH, DH), :],
                                         preferred_element_type=jnp.float32)

                return lax.fori_loop(0, H_PER, head_step,
                                     jnp.zeros((CHUNK, D_MODEL), jnp.float32))

            return lax.cond(c == 0, dense_branch, window_branch, 0)

        def rs_send(s, c):
            return pltpu.make_async_remote_copy(
                src_ref=out_ref.at[row(c), :],
                dst_ref=comm_ref.at[s],
                send_sem=send_sems.at[s],
                recv_sem=recv_sems.at[s],
                device_id=(right,),
                device_id_type=pl.DeviceIdType.MESH,
            )

        def rs_step(s, carry):
            c = mod(p - s)
            val = chunk_partial(c)
            out_ref[row(c), :] = val

            @pl.when(s > 0)
            def _():
                rs_send(s - 1, c).wait_recv()
                out_ref[row(c), :] += comm_ref[s - 1]

            @pl.when(s < N_DEV - 1)
            def _():
                rs_send(s, c).start()

            return carry

        lax.fori_loop(0, N_DEV, rs_step, 0)
        for s in range(N_HOP):
            rs_send(s, mod(p - s)).wait_send()

        own = mod(p + 1)
        ag_ref[own] = out_ref[row(own), :].astype(jnp.bfloat16)

        def ag_copy(s):
            g = mod(p + 1 - s)
            return pltpu.make_async_remote_copy(
                src_ref=ag_ref.at[g],
                dst_ref=ag_ref.at[g],
                send_sem=send_sems.at[N_HOP + s],
                recv_sem=recv_sems.at[N_HOP + s],
                device_id=(right,),
                device_id_type=pl.DeviceIdType.MESH,
            )

        def ag_recv(s):
            g = mod(p - s)
            return pltpu.make_async_remote_copy(
                src_ref=ag_ref.at[g],
                dst_ref=ag_ref.at[g],
                send_sem=send_sems.at[N_HOP + s],
                recv_sem=recv_sems.at[N_HOP + s],
                device_id=(left,),
                device_id_type=pl.DeviceIdType.MESH,
            )

        ag_copy(0).start()
        for s in range(1, N_HOP):
            ag_recv(s - 1).wait_recv()
            ag_copy(s).start()
            g = mod(p - s + 1)
            out_ref[row(g), :] = ag_ref[g].astype(jnp.float32)
        ag_recv(N_HOP - 1).wait_recv()
        g = mod(p - N_HOP + 1)
        out_ref[row(g), :] = ag_ref[g].astype(jnp.float32)
        for s in range(N_HOP):
            ag_copy(s).wait_send()

    out = pl.pallas_call(
        body,
        out_shape=jax.ShapeDtypeStruct((SQ, D_MODEL), jnp.float32),
        in_specs=[pl.BlockSpec(memory_space=pltpu.VMEM)] * 5,
        out_specs=pl.BlockSpec(memory_space=pltpu.VMEM),
        scratch_shapes=[
            pltpu.VMEM((N_HOP, CHUNK, D_MODEL), jnp.float32),
            pltpu.VMEM((N_DEV, CHUNK, D_MODEL), jnp.bfloat16),
            pltpu.SemaphoreType.DMA((2 * N_HOP,)),
            pltpu.SemaphoreType.DMA((2 * N_HOP,)),
        ],
        compiler_params=pltpu.CompilerParams(
            collective_id=0,
            vmem_limit_bytes=100 * 1024 * 1024,
        ),
    )(x2, wq_s, k, v, wo_s)
    return out[None]


# device time: 221432 ns/iter; 1.7639x vs baseline; 1.1221x over previous
import jax
import jax.numpy as jnp
from jax import lax
from jax.experimental import pallas as pl
from jax.experimental.pallas import tpu as pltpu

N_DEV = 16
SQ = 2048
SKV = 2048
H_PER = 8
DH = 128
D_MODEL = 1024
D_HEADS = H_PER * DH
SCALE = 0.08838834764831843
CHUNK = SQ // N_DEV
N_HOP = N_DEV - 1
WIN = 384
GLB = 128


def kernel(x, Wq, K_ext, V_ext, Wo):
    pos = lax.axis_index("i")
    wq_s = lax.dynamic_slice(Wq, (0, pos * D_HEADS),
                             (D_MODEL, D_HEADS)).astype(jnp.bfloat16)
    wo_s = lax.dynamic_slice(Wo, (pos * D_HEADS, 0),
                             (D_HEADS, D_MODEL)).astype(jnp.bfloat16)
    x2 = x[0].astype(jnp.bfloat16)
    k = jnp.transpose(K_ext[0], (1, 0, 2)).astype(jnp.bfloat16)
    v = jnp.transpose(V_ext[0], (1, 0, 2)).astype(jnp.bfloat16)

    def body(x_ref, wq_ref, k_ref, v_ref, wo_ref, out_ref,
             comm_ref, ag_ref, send_sems, recv_sems):
        p = lax.axis_index("i")
        left = lax.rem(p + N_DEV - 1, N_DEV)
        right = lax.rem(p + 1, N_DEV)

        def mod(a):
            return lax.rem(a + 2 * N_DEV, N_DEV)

        def row(c):
            return pl.ds(c * CHUNK, CHUNK)

        barrier_sem = pltpu.get_barrier_semaphore()
        for nbr in (left, right):
            pl.semaphore_signal(barrier_sem, inc=1, device_id=(nbr,),
                                device_id_type=pl.DeviceIdType.MESH)
        pl.semaphore_wait(barrier_sem, 2)

        def chunk_partial(c):
            r0 = c * CHUNK
            x_blk = x_ref[row(c), :]
            qi = lax.broadcasted_iota(jnp.int32, (CHUNK, SKV), 0) + r0

            def qh_of(h):
                return jnp.dot(x_blk, wq_ref[:, pl.ds(h * DH, DH)],
                               preferred_element_type=jnp.float32
                               ).astype(jnp.bfloat16)

            def dense_branch(_):
                ki = lax.broadcasted_iota(jnp.int32, (CHUNK, SKV), 1)
                mask = (jnp.abs(qi - ki) <= 128) | (ki < 32) | (qi < 32)

                def head_step(h, acc):
                    scores = lax.dot_general(
                        qh_of(h), k_ref[h], (((1,), (1,)), ((), ())),
                        preferred_element_type=jnp.float32) * SCALE
                    scores = jnp.where(mask, scores, jnp.float32(-1e9))
                    m = jnp.max(scores, axis=1, keepdims=True)
                    w = jnp.exp(scores - m)
                    w = w / jnp.sum(w, axis=1, keepdims=True)
                    ctx = jnp.dot(w.astype(jnp.bfloat16), v_ref[h],
                                  preferred_element_type=jnp.float32)
                    return acc + jnp.dot(ctx.astype(jnp.bfloat16),
                                         wo_ref[pl.ds(h * DH, DH), :],
                                         preferred_element_type=jnp.float32)

                return lax.fori_loop(0, H_PER, head_step,
                                     jnp.zeros((CHUNK, D_MODEL), jnp.float32))

            def window_branch(_):
                ws = lax.min((c - 1) * CHUNK, SKV - WIN)
                qi_w = qi[:, :WIN]
                ki_w = lax.broadcasted_iota(jnp.int32, (CHUNK, WIN), 1) + ws
                mask_w = (jnp.abs(qi_w - ki_w) <= 128) | (ki_w < 32)
                kg = lax.broadcasted_iota(jnp.int32, (CHUNK, GLB), 1)
                mask_g = (kg < 32) & (kg < ws)

                def head_step(h, acc):
                    qh = qh_of(h)
                    k_win = k_ref[h, pl.ds(ws, WIN), :]
                    sw = lax.dot_general(
                        qh, k_win, (((1,), (1,)), ((), ())),
                        preferred_element_type=jnp.float32) * SCALE
                    sg = lax.dot_general(
                        qh, k_ref[h, :GLB, :], (((1,), (1,)), ((), ())),
                        preferred_element_type=jnp.float32) * SCALE
                    sw = jnp.where(mask_w, sw, jnp.float32(-1e9))
                    sg = jnp.where(mask_g, sg, jnp.float32(-1e9))
                    m = jnp.maximum(jnp.max(sw, axis=1, keepdims=True),
                                    jnp.max(sg, axis=1, keepdims=True))
                    ww = jnp.exp(sw - m)
                    wg = jnp.exp(sg - m)
                    denom = (jnp.sum(ww, axis=1, keepdims=True)
                             + jnp.sum(wg, axis=1, keepdims=True))
                    ctx = (jnp.dot(ww.astype(jnp.bfloat16),
                                   v_ref[h, pl.ds(ws, WIN), :],
                                   preferred_element_type=jnp.float32)
                           + jnp.dot(wg.astype(jnp.bfloat16),
                                     v_ref[h, :GLB, :],
                                     preferred_element_type=jnp.float32))
                    ctx = ctx / denom
                    return acc + jnp.dot(ctx.astype(jnp.bfloat16),
                                         wo_ref[pl.ds(h * DH, DH), :],
                                         preferred_element_type=jnp.float32)

                return lax.fori_loop(0, H_PER, head_step,
                                     jnp.zeros((CHUNK, D_MODEL), jnp.float32))

            return lax.cond(c == 0, dense_branch, window_branch, 0)


        def remote(src, dst, sem, target):
            return pltpu.make_async_remote_copy(
                src_ref=src, dst_ref=dst,
                send_sem=send_sems.at[sem], recv_sem=recv_sems.at[sem],
                device_id=(target,), device_id_type=pl.DeviceIdType.MESH,
            )

        def rsr(s):
            return remote(out_ref.at[row(mod(p + 8 - s)), :],
                          comm_ref.at[s], s, right)

        def rsl(s):
            return remote(out_ref.at[row(mod(p - 7 + s)), :],
                          comm_ref.at[8 + s], 8 + s, left)

        out_ref[row(p), :] = chunk_partial(p)

        def rs_step(s, carry):
            cr = mod(p + 8 - s)
            out_ref[row(cr), :] = chunk_partial(cr)

            @pl.when(s > 0)
            def _():
                rsr(s - 1).wait_recv()
                out_ref[row(cr), :] += comm_ref[s - 1]

            rsr(s).start()

            @pl.when(s < 7)
            def _():
                cl = mod(p - 7 + s)
                out_ref[row(cl), :] = chunk_partial(cl)

            @pl.when((s > 0) & (s < 7))
            def _():
                rsl(s - 1).wait_recv()
                out_ref[row(mod(p - 7 + s)), :] += comm_ref[8 + s - 1]

            @pl.when(s < 7)
            def _():
                rsl(s).start()

            return carry

        lax.fori_loop(0, 8, rs_step, 0)
        rsr(7).wait_recv()
        out_ref[row(p), :] += comm_ref[7]
        rsl(6).wait_recv()
        out_ref[row(p), :] += comm_ref[14]
        for s in range(8):
            rsr(s).wait_send()
        for s in range(7):
            rsl(s).wait_send()

        ag_ref[p] = out_ref[row(p), :].astype(jnp.bfloat16)

        def ag(c, sem, target):
            return remote(ag_ref.at[c], ag_ref.at[c], sem, target)

        def store_f32(g):
            out_ref[row(g), :] = ag_ref[g].astype(jnp.float32)

        ag(p, 15, right).start()
        ag(p, 23, left).start()
        for s in range(1, 8):
            ag(mod(p - s), 15 + s - 1, left).wait_recv()
            ag(mod(p - s), 15 + s, right).start()
            store_f32(mod(p - s))
            if s < 7:
                ag(mod(p + s), 23 + s - 1, right).wait_recv()
                ag(mod(p + s), 23 + s, left).start()
                store_f32(mod(p + s))
        ag(mod(p - 8), 22, left).wait_recv()
        store_f32(mod(p - 8))
        ag(mod(p + 7), 29, right).wait_recv()
        store_f32(mod(p + 7))
        for s in range(8):
            ag(mod(p - s), 15 + s, right).wait_send()
        for s in range(7):
            ag(mod(p + s), 23 + s, left).wait_send()

    out = pl.pallas_call(
        body,
        out_shape=jax.ShapeDtypeStruct((SQ, D_MODEL), jnp.float32),
        in_specs=[pl.BlockSpec(memory_space=pltpu.VMEM)] * 5,
        out_specs=pl.BlockSpec(memory_space=pltpu.VMEM),
        scratch_shapes=[
            pltpu.VMEM((N_HOP, CHUNK, D_MODEL), jnp.float32),
            pltpu.VMEM((N_DEV, CHUNK, D_MODEL), jnp.bfloat16),
            pltpu.SemaphoreType.DMA((2 * N_HOP,)),
            pltpu.SemaphoreType.DMA((2 * N_HOP,)),
        ],
        compiler_params=pltpu.CompilerParams(
            collective_id=0,
            vmem_limit_bytes=100 * 1024 * 1024,
        ),
    )(x2, wq_s, k, v, wo_s)
    return out[None]


# device time: 202818 ns/iter; 1.9258x vs baseline; 1.0918x over previous
import jax
import jax.numpy as jnp
from jax import lax
from jax.experimental import pallas as pl
from jax.experimental.pallas import tpu as pltpu

N_DEV = 16
SQ = 2048
SKV = 2048
H_PER = 8
DH = 128
D_MODEL = 1024
D_HEADS = H_PER * DH
SCALE = 0.08838834764831843
CHUNK = SQ // N_DEV
N_HOP = N_DEV - 1
WIN = 384
GLB = 128


def kernel(x, Wq, K_ext, V_ext, Wo):
    pos = lax.axis_index("i")
    wq_s = lax.dynamic_slice(Wq, (0, pos * D_HEADS),
                             (D_MODEL, D_HEADS)).astype(jnp.bfloat16)
    wo_s = lax.dynamic_slice(Wo, (pos * D_HEADS, 0),
                             (D_HEADS, D_MODEL)).astype(jnp.bfloat16)
    x2 = x[0].astype(jnp.bfloat16)
    k = jnp.transpose(K_ext[0], (1, 0, 2)).astype(jnp.bfloat16)
    v_t = jnp.transpose(V_ext[0], (1, 0, 2)).astype(jnp.bfloat16)
    v = jnp.concatenate(
        [v_t,
         jnp.ones((H_PER, SKV, 1), jnp.bfloat16),
         jnp.zeros((H_PER, SKV, DH - 1), jnp.bfloat16)], axis=2)

    def body(x_ref, wq_ref, k_ref, v_ref, wo_ref, out_ref,
             comm_ref, ag_ref, q_ref, send_sems, recv_sems):
        p = lax.axis_index("i")
        left = lax.rem(p + N_DEV - 1, N_DEV)
        right = lax.rem(p + 1, N_DEV)

        def mod(a):
            return lax.rem(a + 2 * N_DEV, N_DEV)

        def row(c):
            return pl.ds(c * CHUNK, CHUNK)

        barrier_sem = pltpu.get_barrier_semaphore()
        for nbr in (left, right):
            pl.semaphore_signal(barrier_sem, inc=1, device_id=(nbr,),
                                device_id_type=pl.DeviceIdType.MESH)
        pl.semaphore_wait(barrier_sem, 2)

        def chunk_partial(c):
            r0 = c * CHUNK
            x_blk = x_ref[row(c), :]
            qi = lax.broadcasted_iota(jnp.int32, (CHUNK, SKV), 0) + r0
            q_ref[...] = jnp.dot(
                x_blk, wq_ref[...],
                preferred_element_type=jnp.float32).astype(jnp.bfloat16)

            def finish(h, aug, acc):
                ctx = aug[:, :DH] * (1.0 / aug[:, DH:DH + 1])
                return acc + jnp.dot(ctx.astype(jnp.bfloat16),
                                     wo_ref[pl.ds(h * DH, DH), :],
                                     preferred_element_type=jnp.float32)

            def dense_branch(_):
                ki = lax.broadcasted_iota(jnp.int32, (CHUNK, SKV), 1)
                mask = (jnp.abs(qi - ki) <= 128) | (ki < 32) | (qi < 32)

                def head_step(h, acc):
                    qh = q_ref[:, pl.ds(h * DH, DH)]
                    scores = lax.dot_general(
                        qh, k_ref[h], (((1,), (1,)), ((), ())),
                        preferred_element_type=jnp.float32) * SCALE
                    w = jnp.exp(jnp.where(mask, scores, jnp.float32(-1e9)))
                    aug = jnp.dot(w.astype(jnp.bfloat16), v_ref[h],
                                  preferred_element_type=jnp.float32)
                    return finish(h, aug, acc)

                return lax.fori_loop(0, H_PER, head_step,
                                     jnp.zeros((CHUNK, D_MODEL), jnp.float32))

            def window_branch(_):
                ws = lax.min((c - 1) * CHUNK, SKV - WIN)
                qi_w = qi[:, :WIN]
                ki_w = lax.broadcasted_iota(jnp.int32, (CHUNK, WIN), 1) + ws
                mask_w = (jnp.abs(qi_w - ki_w) <= 128) | (ki_w < 32)
                kg = lax.broadcasted_iota(jnp.int32, (CHUNK, GLB), 1)
                mask_g = (kg < 32) & (kg < ws)

                def head_step(h, acc):
                    qh = q_ref[:, pl.ds(h * DH, DH)]
                    k_win = k_ref[h, pl.ds(ws, WIN), :]
                    sw = lax.dot_general(
                        qh, k_win, (((1,), (1,)), ((), ())),
                        preferred_element_type=jnp.float32) * SCALE
                    sg = lax.dot_general(
                        qh, k_ref[h, :GLB, :], (((1,), (1,)), ((), ())),
                        preferred_element_type=jnp.float32) * SCALE
                    ww = jnp.exp(jnp.where(mask_w, sw, jnp.float32(-1e9)))
                    wg = jnp.exp(jnp.where(mask_g, sg, jnp.float32(-1e9)))
                    aug = (jnp.dot(ww.astype(jnp.bfloat16),
                                   v_ref[h, pl.ds(ws, WIN), :],
                                   preferred_element_type=jnp.float32)
                           + jnp.dot(wg.astype(jnp.bfloat16),
                                     v_ref[h, :GLB, :],
                                     preferred_element_type=jnp.float32))
                    return finish(h, aug, acc)

                return lax.fori_loop(0, H_PER, head_step,
                                     jnp.zeros((CHUNK, D_MODEL), jnp.float32))

            return lax.cond(c == 0, dense_branch, window_branch, 0)


        def remote(src, dst, sem, target):
            return pltpu.make_async_remote_copy(
                src_ref=src, dst_ref=dst,
                send_sem=send_sems.at[sem], recv_sem=recv_sems.at[sem],
                device_id=(target,), device_id_type=pl.DeviceIdType.MESH,
            )

        def rsr(s):
            return remote(out_ref.at[row(mod(p + 8 - s)), :],
                          comm_ref.at[s], s, right)

        def rsl(s):
            return remote(out_ref.at[row(mod(p - 7 + s)), :],
                          comm_ref.at[8 + s], 8 + s, left)

        out_ref[row(p), :] = chunk_partial(p)

        def rs_step(s, carry):
            cr = mod(p + 8 - s)
            out_ref[row(cr), :] = chunk_partial(cr)

            @pl.when(s > 0)
            def _():
                rsr(s - 1).wait_recv()
                out_ref[row(cr), :] += comm_ref[s - 1]

            rsr(s).start()

            @pl.when(s < 7)
            def _():
                cl = mod(p - 7 + s)
                out_ref[row(cl), :] = chunk_partial(cl)

            @pl.when((s > 0) & (s < 7))
            def _():
                rsl(s - 1).wait_recv()
                out_ref[row(mod(p - 7 + s)), :] += comm_ref[8 + s - 1]

            @pl.when(s < 7)
            def _():
                rsl(s).start()

            return carry

        lax.fori_loop(0, 8, rs_step, 0)
        rsr(7).wait_recv()
        out_ref[row(p), :] += comm_ref[7]
        rsl(6).wait_recv()
        out_ref[row(p), :] += comm_ref[14]
        for s in range(8):
            rsr(s).wait_send()
        for s in range(7):
            rsl(s).wait_send()

        ag_ref[p] = out_ref[row(p), :].astype(jnp.bfloat16)

        def ag(c, sem, target):
            return remote(ag_ref.at[c], ag_ref.at[c], sem, target)

        def store_f32(g):
            out_ref[row(g), :] = ag_ref[g].astype(jnp.float32)

        ag(p, 15, right).start()
        ag(p, 23, left).start()
        for s in range(1, 8):
            ag(mod(p - s), 15 + s - 1, left).wait_recv()
            ag(mod(p - s), 15 + s, right).start()
            store_f32(mod(p - s))
            if s < 7:
                ag(mod(p + s), 23 + s - 1, right).wait_recv()
                ag(mod(p + s), 23 + s, left).start()
                store_f32(mod(p + s))
        ag(mod(p - 8), 22, left).wait_recv()
        store_f32(mod(p - 8))
        ag(mod(p + 7), 29, right).wait_recv()
        store_f32(mod(p + 7))
        for s in range(8):
            ag(mod(p - s), 15 + s, right).wait_send()
        for s in range(7):
            ag(mod(p + s), 23 + s, left).wait_send()

    out = pl.pallas_call(
        body,
        out_shape=jax.ShapeDtypeStruct((SQ, D_MODEL), jnp.float32),
        in_specs=[pl.BlockSpec(memory_space=pltpu.VMEM)] * 5,
        out_specs=pl.BlockSpec(memory_space=pltpu.VMEM),
        scratch_shapes=[
            pltpu.VMEM((N_HOP, CHUNK, D_MODEL), jnp.float32),
            pltpu.VMEM((N_DEV, CHUNK, D_MODEL), jnp.bfloat16),
            pltpu.VMEM((CHUNK, D_HEADS), jnp.bfloat16),
            pltpu.SemaphoreType.DMA((2 * N_HOP,)),
            pltpu.SemaphoreType.DMA((2 * N_HOP,)),
        ],
        compiler_params=pltpu.CompilerParams(
            collective_id=0,
            vmem_limit_bytes=100 * 1024 * 1024,
        ),
    )(x2, wq_s, k, v, wo_s)
    return out[None]
